# Initial kernel scaffold; baseline (speedup 1.0000x reference)
import jax
import jax.numpy as jnp
from jax.experimental import pallas as pl


def kernel(x, edge_index, Ws, bs):
    raise NotImplementedError("write your pallas kernel here")


# speedup vs baseline: 4.8456x; 4.8456x over previous
import functools

import jax
import jax.numpy as jnp
from jax import lax
from jax.experimental import pallas as pl
from jax.experimental.pallas import tpu as pltpu
from jax.experimental.pallas import tpu_sc as plsc

N = 10000
E = 320000
NCORE = 2
NSUB = 16
NW = NCORE * NSUB
B = 128
NB = 79
EPT = NB * B
EPAD = EPT * NW
NPAD = 10240
CP = 128
RB = 2000
NRB = N // RB

_mesh = functools.partial(
    plsc.VectorSubcoreMesh, core_axis_name="c", subcore_axis_name="s"
)


def _sc_scatter(nc, srcs, dst3, gflat, zeros_c):
    def body(srcs_r, dst3_r, g_r, z_r, p_r, srcv, dstv, rows, acc, sem):
        core = lax.axis_index("c")
        sub = lax.axis_index("s")
        w = core * NSUB + sub
        pltpu.sync_copy(dst3_r.at[w], dstv)
        for c in range(nc):
            pltpu.sync_copy(srcs_r.at[c, w], srcv)
            pltpu.sync_copy(z_r, acc.at[pl.ds(sub * 640, 640)])
            plsc.subcore_barrier()

            @pl.loop(0, NB)
            def _(j):
                pltpu.async_copy(g_r.at[srcv.at[j]], rows, sem).wait()
                pltpu.sync_copy(rows, acc.at[dstv.at[j]], add=True)

            plsc.subcore_barrier()
            pltpu.sync_copy(
                acc.at[pl.ds(sub * 624, 624)],
                p_r.at[c, core, pl.ds(sub * 624, 624)],
            )

            @pl.when(sub == 15)
            def _():
                pltpu.sync_copy(
                    acc.at[pl.ds(9984, 16)],
                    p_r.at[c, core, pl.ds(9984, 16)],
                )

            plsc.subcore_barrier()

    return pl.kernel(
        body,
        out_type=jax.ShapeDtypeStruct((nc, NCORE, N, CP), jnp.float32),
        mesh=_mesh(),
        scratch_types=[
            pltpu.VMEM((NB, B), jnp.int32),
            pltpu.VMEM((NB, B), jnp.int32),
            pltpu.VMEM((B, CP), jnp.float32),
            pltpu.VMEM_SHARED((NPAD, CP), jnp.float32),
            pltpu.SemaphoreType.DMA,
        ],
    )(srcs, dst3, gflat, zeros_c)


def _tc_dis(degp):
    def body(d_ref, o_ref):
        o_ref[...] = lax.rsqrt(d_ref[0] + d_ref[1] + 1.0)

    return pl.pallas_call(
        body,
        out_shape=jax.ShapeDtypeStruct((N, CP), jnp.float32),
    )(degp)


def _tc_matmul(h3, W3, dis):
    nci, _, _ = h3.shape
    nco, din_pad, _ = W3.shape

    def body(h_ref, w_ref, dis_ref, o_ref):
        acc = jnp.zeros((RB, CP), jnp.float32)
        for i in range(nci):
            acc += jnp.dot(
                h_ref[i],
                w_ref[0, i * CP : (i + 1) * CP, :],
                preferred_element_type=jnp.float32,
            )
        o_ref[0] = dis_ref[...] * acc

    return pl.pallas_call(
        body,
        grid=(NRB, nco),
        in_specs=[
            pl.BlockSpec((nci, RB, CP), lambda r, c: (0, r, 0)),
            pl.BlockSpec((1, din_pad, CP), lambda r, c: (c, 0, 0)),
            pl.BlockSpec((RB, 1), lambda r, c: (r, 0)),
        ],
        out_specs=pl.BlockSpec((1, RB, CP), lambda r, c: (c, r, 0)),
        out_shape=jax.ShapeDtypeStruct((nco, N, CP), jnp.float32),
    )(h3, W3, dis)


def _tc_combine(P, g3, dis, b3, skip3):
    nc = P.shape[0]
    has_skip = skip3 is not None

    def body(p_ref, g_ref, dis_ref, b_ref, *rest):
        if has_skip:
            s_ref, o_ref = rest
        else:
            (o_ref,) = rest
        v = p_ref[0, 0] + p_ref[0, 1] + g_ref[0]
        v = dis_ref[...] * v + b_ref[0]
        if has_skip:
            v += s_ref[0]
        o_ref[0] = jnp.maximum(v, 0.0)

    in_specs = [
        pl.BlockSpec((1, 2, RB, CP), lambda c, r: (c, 0, r, 0)),
        pl.BlockSpec((1, RB, CP), lambda c, r: (c, r, 0)),
        pl.BlockSpec((RB, 1), lambda c, r: (r, 0)),
        pl.BlockSpec((1, 1, CP), lambda c, r: (c, 0, 0)),
    ]
    args = [P, g3, dis, b3]
    if has_skip:
        in_specs.append(pl.BlockSpec((1, RB, CP), lambda c, r: (c, r, 0)))
        args.append(skip3)

    return pl.pallas_call(
        body,
        grid=(nc, NRB),
        in_specs=in_specs,
        out_specs=pl.BlockSpec((1, RB, CP), lambda c, r: (c, r, 0)),
        out_shape=jax.ShapeDtypeStruct((nc, N, CP), jnp.float32),
    )(*args)


def kernel(x, edge_index, Ws, bs):
    src = edge_index[0]
    dst = edge_index[1]
    pad = EPAD - E
    src_p = jnp.concatenate([src, jnp.zeros((pad,), jnp.int32)])
    dst_p = jnp.concatenate([dst, jnp.full((pad,), N, jnp.int32)])
    src3 = src_p.reshape(NW, NB, B)
    dst3 = dst_p.reshape(NW, NB, B)

    zeros_c = jnp.zeros((NPAD // NSUB, CP), jnp.float32)

    srcs_by_nc = {}
    for d in (128, 640, 320, 160, 80, 40):
        nc = -(-d // CP)
        if nc not in srcs_by_nc:
            srcs_by_nc[nc] = jnp.stack([src3 + c * N for c in range(nc)])

    degp = _sc_scatter(1, srcs_by_nc[1], dst3, jnp.ones((N, CP), jnp.float32),
                       zeros_c)
    dis = _tc_dis(degp.reshape(NCORE, N, CP))[:, :1]

    acts = {}
    h3 = x.reshape(1, N, CP)
    for k in range(10):
        W = Ws[k]
        b = bs[k]
        din, dout = W.shape
        nci = h3.shape[0]
        nco = -(-dout // CP)
        Wp = jnp.zeros((nci * CP, nco * CP), jnp.float32)
        Wp = Wp.at[:din, :dout].set(W)
        W3 = Wp.reshape(nci * CP, nco, CP).transpose(1, 0, 2)
        g3 = _tc_matmul(h3, W3, dis)
        gflat = g3.reshape(nco * N, CP)
        P = _sc_scatter(nco, srcs_by_nc[nco], dst3, gflat, zeros_c)
        b3 = jnp.zeros((nco * CP,), jnp.float32).at[:dout].set(b)
        b3 = b3.reshape(nco, 1, CP)
        skip3 = acts.get(9 - k) if 5 <= k <= 8 else None
        h3 = _tc_combine(P, g3, dis, b3, skip3)
        if k <= 3:
            acts[k + 1] = h3

    return h3.reshape(N, CP)
